# Initial kernel scaffold; baseline (speedup 1.0000x reference)
import jax
import jax.numpy as jnp
from jax.experimental import pallas as pl


def kernel(feature, edge_index, edge_weight, W, b):
    raise NotImplementedError("write your pallas kernel here")


# speedup vs baseline: 5.2208x; 5.2208x over previous
import functools

import jax
import jax.numpy as jnp
from jax import lax
from jax.experimental import pallas as pl
from jax.experimental.pallas import tpu as pltpu
from jax.experimental.pallas import tpu_sc as plsc

NSUB = 16
NCORE = 2
LANES = 16
CHUNK = 128


def _make_sc_scatter(n_acc, dh, nch):
    rows_per_sub = n_acc // NSUB
    mesh = plsc.VectorSubcoreMesh(core_axis_name="c", subcore_axis_name="s")

    @functools.partial(
        pl.kernel,
        mesh=mesh,
        compiler_params=pltpu.CompilerParams(use_tc_tiling_on_sc=False),
        out_type=jax.ShapeDtypeStruct((NCORE, n_acc, dh), jnp.float32),
        scratch_types=[
            pltpu.VMEM((nch, CHUNK), jnp.int32),
            pltpu.VMEM((nch, CHUNK), jnp.int32),
            pltpu.VMEM((CHUNK * LANES,), jnp.float32),
            pltpu.VMEM((CHUNK * LANES,), jnp.float32),
            pltpu.VMEM((CHUNK, dh), jnp.float32),
            pltpu.VMEM((CHUNK, dh), jnp.float32),
            pltpu.VMEM_SHARED((n_acc, dh), jnp.float32),
            pltpu.SemaphoreType.DMA,
            pltpu.SemaphoreType.DMA,
        ],
    )
    def sc_scatter(f2_hbm, src_hbm, dst_hbm, w_hbm, out_hbm,
                   src_v, dst_v, wbuf0, wbuf1, buf0, buf1, acc, sem0, sem1):
        c = lax.axis_index("c")
        s = lax.axis_index("s")
        bufs = (buf0, buf1)
        wbufs = (wbuf0, wbuf1)
        sems = (sem0, sem1)

        pltpu.sync_copy(src_hbm.at[c, s], src_v)
        pltpu.sync_copy(dst_hbm.at[s], dst_v)

        def _zrow(r, carry):
            for j in range(dh // LANES):
                buf0[r, pl.ds(LANES * j, LANES)] = jnp.zeros((LANES,), jnp.float32)
            return carry

        lax.fori_loop(0, CHUNK, _zrow, 0)
        for t in range(rows_per_sub // CHUNK):
            r0 = s * rows_per_sub + t * CHUNK
            pltpu.sync_copy(buf0, acc.at[pl.ds(r0, CHUNK)])
        plsc.subcore_barrier()

        pltpu.async_copy(f2_hbm.at[src_v.at[0]], buf0, sem0)
        pltpu.async_copy(w_hbm.at[s * nch], wbuf0, sem0)
        pltpu.async_copy(f2_hbm.at[src_v.at[1]], buf1, sem1)
        pltpu.async_copy(w_hbm.at[s * nch + 1], wbuf1, sem1)

        def _scale(buf, wbuf):
            def _edge(e, carry):
                wvec = wbuf[pl.ds(LANES * e, LANES)]
                for j in range(dh // LANES):
                    sl = pl.ds(LANES * j, LANES)
                    buf[e, sl] = buf[e, sl] * wvec
                return carry

            lax.fori_loop(0, CHUNK, _edge, 0)

        def _pair(i, carry):
            for k in range(2):
                cc = 2 * i + k
                buf, wbuf, sem = bufs[k], wbufs[k], sems[k]
                pltpu.make_async_copy(f2_hbm.at[pl.ds(0, CHUNK)], buf, sem).wait()
                pltpu.make_async_copy(w_hbm.at[0], wbuf, sem).wait()
                _scale(buf, wbuf)
                pltpu.sync_copy(buf, acc.at[dst_v.at[cc]], add=True)

                @pl.when(cc + 2 < nch)
                def _():
                    pltpu.async_copy(f2_hbm.at[src_v.at[cc + 2]], buf, sem)
                    pltpu.async_copy(w_hbm.at[s * nch + cc + 2], wbuf, sem)
            return carry

        lax.fori_loop(0, nch // 2, _pair, 0)

        plsc.subcore_barrier()
        for t in range(rows_per_sub // CHUNK):
            r0 = s * rows_per_sub + t * CHUNK
            pltpu.sync_copy(acc.at[pl.ds(r0, CHUNK)], buf0)
            pltpu.sync_copy(buf0, out_hbm.at[c, pl.ds(r0, CHUNK)])

    return sc_scatter


def _tc_linear(partial, W, b8, n_nodes):
    dh = partial.shape[2]
    d = 2 * dh
    blk = 1000 if n_nodes % 1000 == 0 else n_nodes

    def _body(p_ref, w_ref, b_ref, o_ref):
        y = lax.dot_general(p_ref[0], w_ref[:, 0:dh], (((1,), (1,)), ((), ())),
                            preferred_element_type=jnp.float32)
        y += lax.dot_general(p_ref[1], w_ref[:, dh:d], (((1,), (1,)), ((), ())),
                             preferred_element_type=jnp.float32)
        o_ref[...] = jnp.maximum(y + b_ref[0:1, :], 0.0)

    return pl.pallas_call(
        _body,
        grid=(n_nodes // blk,),
        in_specs=[
            pl.BlockSpec((2, blk, dh), lambda i: (0, i, 0)),
            pl.BlockSpec((d, d), lambda i: (0, 0)),
            pl.BlockSpec((8, d), lambda i: (0, 0)),
        ],
        out_specs=pl.BlockSpec((blk, d), lambda i: (i, 0)),
        out_shape=jax.ShapeDtypeStruct((n_nodes, d), jnp.float32),
    )(partial, W, b8)


def kernel(feature, edge_index, edge_weight, W, b):
    n_nodes, d = feature.shape
    dh = d // 2
    e = edge_index.shape[1]
    per_s = NSUB * CHUNK
    e_pad = ((e + per_s - 1) // per_s) * per_s
    nch = e_pad // per_s
    if nch % 2 == 1:
        nch += 1
        e_pad += per_s

    src = edge_index[0].astype(jnp.int32)
    dst = edge_index[1].astype(jnp.int32)
    w = edge_weight.astype(jnp.float32)
    pad = e_pad - e
    src = jnp.concatenate([src, jnp.zeros((pad,), jnp.int32)])
    dst = jnp.concatenate([dst, jnp.zeros((pad,), jnp.int32)]).reshape(NSUB, nch, CHUNK)
    w = jnp.concatenate([w, jnp.zeros((pad,), jnp.float32)])
    f2 = jnp.concatenate([feature[:, :dh], feature[:, dh:]], axis=0)
    src2 = jnp.stack([src, src + n_nodes]).reshape(NCORE, NSUB, nch, CHUNK)
    w16 = jnp.broadcast_to(w[:, None], (e_pad, LANES)).reshape(NSUB * nch, CHUNK * LANES)

    n_acc = ((n_nodes + NSUB * CHUNK - 1) // (NSUB * CHUNK)) * (NSUB * CHUNK)
    partial = _make_sc_scatter(n_acc, dh, nch)(f2, src2, dst, w16)
    b8 = jnp.broadcast_to(b[None, :], (8, d))
    return _tc_linear(partial, W, b8, n_nodes)


# speedup vs baseline: 6.0981x; 1.1680x over previous
import functools

import jax
import jax.numpy as jnp
from jax import lax
from jax.experimental import pallas as pl
from jax.experimental.pallas import tpu as pltpu
from jax.experimental.pallas import tpu_sc as plsc

NSUB = 16
NCORE = 2
LANES = 16
CHUNK = 128


def _make_sc_scatter(n_acc, dh, nch):
    rows_per_sub = n_acc // NSUB
    mesh = plsc.VectorSubcoreMesh(core_axis_name="c", subcore_axis_name="s")

    @functools.partial(
        pl.kernel,
        mesh=mesh,
        compiler_params=pltpu.CompilerParams(use_tc_tiling_on_sc=False),
        out_type=jax.ShapeDtypeStruct((NCORE, n_acc, dh), jnp.float32),
        scratch_types=[
            pltpu.VMEM((nch, CHUNK), jnp.int32),
            pltpu.VMEM((nch, CHUNK), jnp.int32),
            pltpu.VMEM((CHUNK * LANES,), jnp.float32),
            pltpu.VMEM((CHUNK * LANES,), jnp.float32),
            pltpu.VMEM((CHUNK, dh), jnp.float32),
            pltpu.VMEM((CHUNK, dh), jnp.float32),
            pltpu.VMEM_SHARED((n_acc, dh), jnp.float32),
            pltpu.SemaphoreType.DMA,
            pltpu.SemaphoreType.DMA,
        ],
    )
    def sc_scatter(f2_hbm, src_hbm, dst_hbm, w_hbm, out_hbm,
                   src_v, dst_v, wbuf0, wbuf1, buf0, buf1, acc, sem0, sem1):
        c = lax.axis_index("c")
        s = lax.axis_index("s")
        bufs = (buf0, buf1)
        wbufs = (wbuf0, wbuf1)
        sems = (sem0, sem1)

        pltpu.sync_copy(src_hbm.at[c, s], src_v)
        pltpu.sync_copy(dst_hbm.at[s], dst_v)

        @plsc.parallel_loop(0, CHUNK, 1, unroll=8)
        def _zrow(r):
            for j in range(dh // LANES):
                buf0[r, pl.ds(LANES * j, LANES)] = jnp.zeros((LANES,), jnp.float32)
        for t in range(rows_per_sub // CHUNK):
            r0 = s * rows_per_sub + t * CHUNK
            pltpu.sync_copy(buf0, acc.at[pl.ds(r0, CHUNK)])
        plsc.subcore_barrier()

        pltpu.async_copy(f2_hbm.at[src_v.at[0]], buf0, sem0)
        pltpu.async_copy(w_hbm.at[s * nch], wbuf0, sem0)
        pltpu.async_copy(f2_hbm.at[src_v.at[1]], buf1, sem1)
        pltpu.async_copy(w_hbm.at[s * nch + 1], wbuf1, sem1)

        def _scale(buf, wbuf):
            @plsc.parallel_loop(0, CHUNK, 1, unroll=8)
            def _edge(e):
                wvec = wbuf[pl.ds(LANES * e, LANES)]
                for j in range(dh // LANES):
                    sl = pl.ds(LANES * j, LANES)
                    buf[e, sl] = buf[e, sl] * wvec

        def _pair(i, carry):
            for k in range(2):
                cc = 2 * i + k
                buf, wbuf, sem = bufs[k], wbufs[k], sems[k]
                pltpu.make_async_copy(f2_hbm.at[pl.ds(0, CHUNK)], buf, sem).wait()
                pltpu.make_async_copy(w_hbm.at[0], wbuf, sem).wait()
                _scale(buf, wbuf)
                pltpu.sync_copy(buf, acc.at[dst_v.at[cc]], add=True)

                @pl.when(cc + 2 < nch)
                def _():
                    pltpu.async_copy(f2_hbm.at[src_v.at[cc + 2]], buf, sem)
                    pltpu.async_copy(w_hbm.at[s * nch + cc + 2], wbuf, sem)
            return carry

        lax.fori_loop(0, nch // 2, _pair, 0)

        plsc.subcore_barrier()
        for t in range(rows_per_sub // CHUNK):
            r0 = s * rows_per_sub + t * CHUNK
            pltpu.sync_copy(acc.at[pl.ds(r0, CHUNK)], buf0)
            pltpu.sync_copy(buf0, out_hbm.at[c, pl.ds(r0, CHUNK)])

    return sc_scatter


def _tc_linear(partial, W, b8, n_nodes):
    dh = partial.shape[2]
    d = 2 * dh
    blk = 1000 if n_nodes % 1000 == 0 else n_nodes

    def _body(p_ref, w_ref, b_ref, o_ref):
        y = lax.dot_general(p_ref[0], w_ref[:, 0:dh], (((1,), (1,)), ((), ())),
                            preferred_element_type=jnp.float32)
        y += lax.dot_general(p_ref[1], w_ref[:, dh:d], (((1,), (1,)), ((), ())),
                             preferred_element_type=jnp.float32)
        o_ref[...] = jnp.maximum(y + b_ref[0:1, :], 0.0)

    return pl.pallas_call(
        _body,
        grid=(n_nodes // blk,),
        in_specs=[
            pl.BlockSpec((2, blk, dh), lambda i: (0, i, 0)),
            pl.BlockSpec((d, d), lambda i: (0, 0)),
            pl.BlockSpec((8, d), lambda i: (0, 0)),
        ],
        out_specs=pl.BlockSpec((blk, d), lambda i: (i, 0)),
        out_shape=jax.ShapeDtypeStruct((n_nodes, d), jnp.float32),
    )(partial, W, b8)


def kernel(feature, edge_index, edge_weight, W, b):
    n_nodes, d = feature.shape
    dh = d // 2
    e = edge_index.shape[1]
    per_s = NSUB * CHUNK
    e_pad = ((e + per_s - 1) // per_s) * per_s
    nch = e_pad // per_s
    if nch % 2 == 1:
        nch += 1
        e_pad += per_s

    src = edge_index[0].astype(jnp.int32)
    dst = edge_index[1].astype(jnp.int32)
    w = edge_weight.astype(jnp.float32)
    pad = e_pad - e
    src = jnp.concatenate([src, jnp.zeros((pad,), jnp.int32)])
    dst = jnp.concatenate([dst, jnp.zeros((pad,), jnp.int32)]).reshape(NSUB, nch, CHUNK)
    w = jnp.concatenate([w, jnp.zeros((pad,), jnp.float32)])
    f2 = jnp.concatenate([feature[:, :dh], feature[:, dh:]], axis=0)
    src2 = jnp.stack([src, src + n_nodes]).reshape(NCORE, NSUB, nch, CHUNK)
    w16 = jnp.broadcast_to(w[:, None], (e_pad, LANES)).reshape(NSUB * nch, CHUNK * LANES)

    n_acc = ((n_nodes + NSUB * CHUNK - 1) // (NSUB * CHUNK)) * (NSUB * CHUNK)
    partial = _make_sc_scatter(n_acc, dh, nch)(f2, src2, dst, w16)
    b8 = jnp.broadcast_to(b[None, :], (8, d))
    return _tc_linear(partial, W, b8, n_nodes)
